# Initial kernel scaffold; baseline (speedup 1.0000x reference)
import jax
import jax.numpy as jnp
from jax.experimental import pallas as pl


def kernel(belive_map):
    raise NotImplementedError("write your pallas kernel here")


# speedup vs baseline: 4.1901x; 4.1901x over previous
import jax
import jax.numpy as jnp
from jax.experimental import pallas as pl
from jax.experimental.pallas import tpu as pltpu

_MIN_DISTANCE = 3
_THRESHOLD_REL = 0.01
_MAX_NUM_PEAKS = 100
_H = 512
_W = 512
_BIG = 1 << 30


def _nms_channel_kernel(x_ref, scores_ref, gidx_ref, candT_ref):
    x = x_ref[0, 0]

    m = jnp.max(x)
    e = jnp.exp(x - m)
    s = jnp.sum(e)
    p = e / s

    k = 2 * _MIN_DISTANCE + 1
    zpad_r = jnp.zeros((_MIN_DISTANCE, _W), jnp.float32)
    pv = jnp.concatenate([zpad_r, p, zpad_r], axis=0)
    pooled_v = pv[0:_H, :]
    for d in range(1, k):
        pooled_v = jnp.maximum(pooled_v, pv[d:d + _H, :])
    zpad_c = jnp.zeros((_H, _MIN_DISTANCE), jnp.float32)
    ph = jnp.concatenate([zpad_c, pooled_v, zpad_c], axis=1)
    pooled = ph[:, 0:_W]
    for d in range(1, k):
        pooled = jnp.maximum(pooled, ph[:, d:d + _W])

    thr_abs = 1.0 / (_H * _W) * 2.0
    mx = jnp.max(p)
    mask = (pooled == p) & (p > thr_abs) & (p > _THRESHOLD_REL * mx)
    cand = jnp.where(mask, p, jnp.float32(-1.0))

    rows2d = jax.lax.broadcasted_iota(jnp.int32, (_H, _W), 0)
    cmax = jnp.max(cand, axis=0, keepdims=True)
    carg = jnp.min(jnp.where(cand == cmax, rows2d, _BIG), axis=0,
                   keepdims=True)

    candT_ref[...] = cand.T

    lane_w = jax.lax.broadcasted_iota(jnp.int32, (1, _W), 1)
    lane_k = jax.lax.broadcasted_iota(jnp.int32, (1, 128), 1)

    def body(i, st):
        cmax, carg, svec, gvec = st
        mval = jnp.max(cmax)
        g = jnp.min(jnp.where(cmax == mval, carg * _W + lane_w, _BIG))
        r = g // _W
        c = g % _W
        svec = jnp.where(lane_k == i, mval, svec)
        gvec = jnp.where(lane_k == i, g, gvec)
        rowv = candT_ref[pl.ds(c, 1), :]
        rowv = jnp.where(lane_w == r, jnp.float32(-3.0), rowv)
        candT_ref[pl.ds(c, 1), :] = rowv
        nm = jnp.max(rowv)
        na = jnp.min(jnp.where(rowv == nm, lane_w, _BIG))
        cmax = jnp.where(lane_w == c, nm, cmax)
        carg = jnp.where(lane_w == c, na, carg)
        return cmax, carg, svec, gvec

    svec0 = jnp.zeros((1, 128), jnp.float32)
    gvec0 = jnp.zeros((1, 128), jnp.int32)
    _, _, svec, gvec = jax.lax.fori_loop(
        0, _MAX_NUM_PEAKS, body, (cmax, carg, svec0, gvec0))

    scores_ref[0] = svec
    gidx_ref[0] = gvec


def kernel(belive_map):
    B, S, H, W = belive_map.shape
    bs = B * S
    raw_scores, raw_gidx = pl.pallas_call(
        _nms_channel_kernel,
        grid=(bs,),
        in_specs=[pl.BlockSpec((1, 1, H, W), lambda i: (i // S, i % S, 0, 0))],
        out_specs=[
            pl.BlockSpec((1, 1, 128), lambda i: (i, 0, 0)),
            pl.BlockSpec((1, 1, 128), lambda i: (i, 0, 0)),
        ],
        out_shape=[
            jax.ShapeDtypeStruct((bs, 1, 128), jnp.float32),
            jax.ShapeDtypeStruct((bs, 1, 128), jnp.int32),
        ],
        scratch_shapes=[pltpu.VMEM((W, H), jnp.float32)],
        compiler_params=pltpu.CompilerParams(
            dimension_semantics=("arbitrary",)),
    )(belive_map)

    scores_raw = raw_scores[:, 0, :_MAX_NUM_PEAKS].reshape(B, S, _MAX_NUM_PEAKS)
    g = raw_gidx[:, 0, :_MAX_NUM_PEAKS].reshape(B, S, _MAX_NUM_PEAKS)
    valid = scores_raw > 0.0
    scores = jnp.where(valid, scores_raw, 0.0)
    rows = g // W
    cols = g % W
    seg = jnp.broadcast_to(jnp.arange(S, dtype=jnp.int32)[None, :, None],
                           (B, S, _MAX_NUM_PEAKS))
    skeletons = jnp.stack([seg, cols, rows], axis=-1)
    return skeletons, scores, valid
